# initial kernel scaffold (unmeasured)
import jax
import jax.numpy as jnp
from jax import lax
from jax.experimental import pallas as pl
from jax.experimental.pallas import tpu as pltpu

N_DEV = 32
M = 4096
N = 8192
ROWS = M // N_DEV

_sem_signal = getattr(pl, "semaphore_signal", None) or pltpu.semaphore_signal
_sem_wait = getattr(pl, "semaphore_wait", None) or pltpu.semaphore_wait


def _ring_allreduce(partial):

    def body(p_ref, o_ref, local_ref, acc_ref, recv_ref,
             send_sems, recv_sems, dma_sem_a, dma_sem_b, credit):
        my = lax.axis_index("i")
        left = (my - 1) % N_DEV
        right = (my + 1) % N_DEV

        barrier = pltpu.get_barrier_semaphore()
        for nbr in (left, right):
            _sem_signal(barrier, inc=1, device_id=(nbr,),
                        device_id_type=pl.DeviceIdType.MESH)
        _sem_wait(barrier, 2)

        init = pltpu.make_async_copy(
            p_ref.at[pl.ds(my * ROWS, ROWS)], acc_ref.at[0], dma_sem_a)
        init.start()
        init.wait()

        for s in range(2 * (N_DEV - 1)):
            if s >= 2:
                _sem_wait(credit, 1)

            if s <= 31:
                src = acc_ref.at[s % 2]
            else:
                src = recv_ref.at[(s - 1) % 2]

            rdma = pltpu.make_async_remote_copy(
                src_ref=src,
                dst_ref=recv_ref.at[s % 2],
                send_sem=send_sems.at[s % 2],
                recv_sem=recv_sems.at[s % 2],
                device_id=(right,),
                device_id_type=pl.DeviceIdType.MESH,
            )
            rdma.start()

            if s <= 30:
                rc = (my - s - 1) % N_DEV
                lcopy = pltpu.make_async_copy(
                    p_ref.at[pl.ds(rc * ROWS, ROWS)],
                    local_ref.at[s % 2], dma_sem_a)
                lcopy.start()

            rdma.wait()

            if 1 <= s <= 60:
                _sem_signal(credit, inc=1, device_id=(left,),
                            device_id_type=pl.DeviceIdType.MESH)

            if s <= 30:
                lcopy.wait()
                acc_ref[(s + 1) % 2] = recv_ref[s % 2] + local_ref[s % 2]
            else:
                gc = (my - (s - 31)) % N_DEV
                ocopy = pltpu.make_async_copy(
                    recv_ref.at[s % 2],
                    o_ref.at[pl.ds(gc * ROWS, ROWS)], dma_sem_b)
                ocopy.start()
                ocopy.wait()

        ow = (my + 1) % N_DEV
        own = pltpu.make_async_copy(
            acc_ref.at[1], o_ref.at[pl.ds(ow * ROWS, ROWS)], dma_sem_b)
        own.start()
        own.wait()

    return pl.pallas_call(
        body,
        out_shape=jax.ShapeDtypeStruct((M, N), jnp.float32),
        in_specs=[pl.BlockSpec(memory_space=pltpu.ANY)],
        out_specs=pl.BlockSpec(memory_space=pltpu.ANY),
        scratch_shapes=[
            pltpu.VMEM((2, ROWS, N), jnp.float32),
            pltpu.VMEM((2, ROWS, N), jnp.float32),
            pltpu.VMEM((2, ROWS, N), jnp.float32),
            pltpu.SemaphoreType.DMA((2,)),
            pltpu.SemaphoreType.DMA((2,)),
            pltpu.SemaphoreType.DMA,
            pltpu.SemaphoreType.DMA,
            pltpu.SemaphoreType.REGULAR,
        ],
        compiler_params=pltpu.CompilerParams(collective_id=0),
    )(partial)


def kernel(x, w_mat):
    partial = jnp.dot(x, w_mat, preferred_element_type=jnp.float32)

    y = _ring_allreduce(partial)

    amax = jnp.max(jnp.abs(y))
    scale = amax / 127.0
    q = jnp.clip(jnp.round(y / scale), -127.0, 127.0)
    return (q * scale).astype(jnp.float32)


# baseline (device time: 3403352 ns/iter reference)
import jax
import jax.numpy as jnp
from jax import lax
from jax.experimental import pallas as pl
from jax.experimental.pallas import tpu as pltpu

N_DEV = 32
M = 4096
N = 8192
ROWS = M // N_DEV

_sem_signal = getattr(pl, "semaphore_signal", None) or pltpu.semaphore_signal
_sem_wait = getattr(pl, "semaphore_wait", None) or pltpu.semaphore_wait


def _ring_allreduce(partial):

    def body(p_ref, o_ref, local_ref, acc_ref, recv_ref,
             send_sems, recv_sems, dma_sem_a, dma_sem_b, credit):
        my = lax.axis_index("i")
        left = (my - 1) % N_DEV
        right = (my + 1) % N_DEV

        barrier = pltpu.get_barrier_semaphore()
        for nbr in (left, right):
            _sem_signal(barrier, inc=1, device_id=(nbr,),
                        device_id_type=pl.DeviceIdType.MESH)
        _sem_wait(barrier, 2)

        init = pltpu.make_async_copy(
            p_ref.at[pl.ds(my * ROWS, ROWS)], acc_ref.at[0], dma_sem_a)
        init.start()
        init.wait()

        for s in range(2 * (N_DEV - 1)):
            if s >= 2:
                _sem_wait(credit, 1)

            if s <= 31:
                src = acc_ref.at[s % 2]
            else:
                src = recv_ref.at[(s - 1) % 2]

            rdma = pltpu.make_async_remote_copy(
                src_ref=src,
                dst_ref=recv_ref.at[s % 2],
                send_sem=send_sems.at[s % 2],
                recv_sem=recv_sems.at[s % 2],
                device_id=(right,),
                device_id_type=pl.DeviceIdType.MESH,
            )
            rdma.start()

            if s <= 30:
                rc = (my - s - 1) % N_DEV
                lcopy = pltpu.make_async_copy(
                    p_ref.at[pl.ds(rc * ROWS, ROWS)],
                    local_ref.at[s % 2], dma_sem_a)
                lcopy.start()

            rdma.wait()

            if 1 <= s <= 60:
                _sem_signal(credit, inc=1, device_id=(left,),
                            device_id_type=pl.DeviceIdType.MESH)

            if s <= 30:
                lcopy.wait()
                acc_ref[(s + 1) % 2] = recv_ref[s % 2] + local_ref[s % 2]
            else:
                gc = (my - (s - 31)) % N_DEV
                ocopy = pltpu.make_async_copy(
                    recv_ref.at[s % 2],
                    o_ref.at[pl.ds(gc * ROWS, ROWS)], dma_sem_b)
                ocopy.start()
                ocopy.wait()

        ow = (my + 1) % N_DEV
        own = pltpu.make_async_copy(
            acc_ref.at[1], o_ref.at[pl.ds(ow * ROWS, ROWS)], dma_sem_b)
        own.start()
        own.wait()

    return pl.pallas_call(
        body,
        out_shape=jax.ShapeDtypeStruct((M, N), jnp.float32),
        in_specs=[pl.BlockSpec(memory_space=pl.ANY)],
        out_specs=pl.BlockSpec(memory_space=pl.ANY),
        scratch_shapes=[
            pltpu.VMEM((2, ROWS, N), jnp.float32),
            pltpu.VMEM((2, ROWS, N), jnp.float32),
            pltpu.VMEM((2, ROWS, N), jnp.float32),
            pltpu.SemaphoreType.DMA((2,)),
            pltpu.SemaphoreType.DMA((2,)),
            pltpu.SemaphoreType.DMA,
            pltpu.SemaphoreType.DMA,
            pltpu.SemaphoreType.REGULAR,
        ],
        compiler_params=pltpu.CompilerParams(collective_id=0),
    )(partial)


def kernel(x, w_mat):
    partial = jnp.dot(x, w_mat, preferred_element_type=jnp.float32)

    y = _ring_allreduce(partial)

    amax = jnp.max(jnp.abs(y))
    scale = amax / 127.0
    q = jnp.clip(jnp.round(y / scale), -127.0, 127.0)
    return (q * scale).astype(jnp.float32)


# device time: 1323147 ns/iter; 2.5722x vs baseline; 2.5722x over previous
import jax
import jax.numpy as jnp
from jax import lax
from jax.experimental import pallas as pl
from jax.experimental.pallas import tpu as pltpu

N_DEV = 32
M = 4096
N = 8192
ROWS = M // N_DEV
HALF = ROWS // 2

_sem_signal = getattr(pl, "semaphore_signal", None) or pltpu.semaphore_signal
_sem_wait = getattr(pl, "semaphore_wait", None) or pltpu.semaphore_wait
_MESH = pl.DeviceIdType.MESH


def _fused_allreduce_quant(partial):
    def body(p_ref, o_ref,
             local_r, local_l, acc_r, acc_l, sb_r, sb_l, rv_r, rv_l,
             ag_r, ag_l, oq_r, oq_l, dq_r, dq_l, aval, ainb,
             ss_r, rs_r, ss_l, rs_l,
             ag_ss_r, ag_rs_r, ag_ss_l, ag_rs_l,
             pf_sem_r, pf_sem_l, st_sem_r, st_sem_l,
             am_ss, am_rs,
             cr_r, cr_l, acr_r, acr_l):
        my = lax.axis_index("i")
        left = (my - 1) % N_DEV
        right = (my + 1) % N_DEV

        def r_rows(c):
            return pl.ds(c * ROWS, HALF)

        def l_rows(c):
            return pl.ds(c * ROWS + HALF, HALF)

        barrier = pltpu.get_barrier_semaphore()
        for nbr in (left, right):
            _sem_signal(barrier, inc=1, device_id=(nbr,),
                        device_id_type=_MESH)
        _sem_wait(barrier, 2)

        init_r = pltpu.make_async_copy(p_ref.at[r_rows(my)],
                                       acc_r.at[0], pf_sem_r)
        init_l = pltpu.make_async_copy(p_ref.at[l_rows(my)],
                                       acc_l.at[0], pf_sem_l)
        init_r.start()
        init_l.start()
        init_r.wait()
        init_l.wait()
        sb_r[0] = acc_r[0].astype(jnp.bfloat16)
        sb_l[0] = acc_l[0].astype(jnp.bfloat16)

        for s in range(N_DEV - 1):
            cur, nxt = s % 2, (s + 1) % 2
            if s >= 2:
                _sem_wait(cr_r, 1)
                _sem_wait(cr_l, 1)
            rd_r = pltpu.make_async_remote_copy(
                src_ref=sb_r.at[cur], dst_ref=rv_r.at[cur],
                send_sem=ss_r.at[cur], recv_sem=rs_r.at[cur],
                device_id=(right,), device_id_type=_MESH)
            rd_l = pltpu.make_async_remote_copy(
                src_ref=sb_l.at[cur], dst_ref=rv_l.at[cur],
                send_sem=ss_l.at[cur], recv_sem=rs_l.at[cur],
                device_id=(left,), device_id_type=_MESH)
            rd_r.start()
            rd_l.start()

            rc_r = (my - s - 1) % N_DEV
            rc_l = (my + s + 1) % N_DEV
            pf_r = pltpu.make_async_copy(p_ref.at[r_rows(rc_r)],
                                         local_r.at[cur], pf_sem_r)
            pf_l = pltpu.make_async_copy(p_ref.at[l_rows(rc_l)],
                                         local_l.at[cur], pf_sem_l)
            pf_r.start()
            pf_l.start()

            rd_r.wait()
            rd_l.wait()
            pf_r.wait()
            pf_l.wait()

            acc_r[nxt] = rv_r[cur].astype(jnp.float32) + local_r[cur]
            acc_l[nxt] = rv_l[cur].astype(jnp.float32) + local_l[cur]
            if s <= N_DEV - 3:
                sb_r[nxt] = acc_r[nxt].astype(jnp.bfloat16)
                sb_l[nxt] = acc_l[nxt].astype(jnp.bfloat16)
            if s <= N_DEV - 4:
                _sem_signal(cr_r, inc=1, device_id=(left,),
                            device_id_type=_MESH)
                _sem_signal(cr_l, inc=1, device_id=(right,),
                            device_id_type=_MESH)


        am_local = jnp.maximum(jnp.max(jnp.abs(acc_r[1])),
                               jnp.max(jnp.abs(acc_l[1])))
        aval[pl.ds(0, 1)] = jnp.broadcast_to(am_local, (1, 128))
        for k in range(5):
            partner = my ^ (1 << k)
            rd = pltpu.make_async_remote_copy(
                src_ref=aval.at[pl.ds(k, 1)], dst_ref=ainb.at[pl.ds(k, 1)],
                send_sem=am_ss.at[k], recv_sem=am_rs.at[k],
                device_id=(partner,), device_id_type=_MESH)
            rd.start()
            rd.wait()
            aval[pl.ds(k + 1, 1)] = jnp.maximum(aval[pl.ds(k, 1)],
                                                ainb[pl.ds(k, 1)])
        scale = jnp.max(aval[pl.ds(5, 1)]) / 127.0

        oq_r[...] = jnp.clip(jnp.round(acc_r[1] / scale),
                             -127.0, 127.0).astype(jnp.int8)
        oq_l[...] = jnp.clip(jnp.round(acc_l[1] / scale),
                             -127.0, 127.0).astype(jnp.int8)
        dq_r[...] = oq_r[...].astype(jnp.float32) * scale
        dq_l[...] = oq_l[...].astype(jnp.float32) * scale
        st_r = pltpu.make_async_copy(dq_r, o_ref.at[r_rows((my + 1) % N_DEV)],
                                     st_sem_r)
        st_l = pltpu.make_async_copy(dq_l, o_ref.at[l_rows((my - 1) % N_DEV)],
                                     st_sem_l)
        st_r.start()
        st_l.start()
        st_r.wait()
        st_l.wait()

        for t in range(N_DEV - 1):
            cur = t % 2
            if t >= 2:
                _sem_wait(acr_r, 1)
                _sem_wait(acr_l, 1)
            src_r = oq_r if t == 0 else ag_r.at[(t - 1) % 2]
            src_l = oq_l if t == 0 else ag_l.at[(t - 1) % 2]
            rd_r = pltpu.make_async_remote_copy(
                src_ref=src_r, dst_ref=ag_r.at[cur],
                send_sem=ag_ss_r.at[cur], recv_sem=ag_rs_r.at[cur],
                device_id=(right,), device_id_type=_MESH)
            rd_l = pltpu.make_async_remote_copy(
                src_ref=src_l, dst_ref=ag_l.at[cur],
                send_sem=ag_ss_l.at[cur], recv_sem=ag_rs_l.at[cur],
                device_id=(left,), device_id_type=_MESH)
            rd_r.start()
            rd_l.start()
            rd_r.wait()
            rd_l.wait()
            if 1 <= t <= N_DEV - 3:
                _sem_signal(acr_r, inc=1, device_id=(left,),
                            device_id_type=_MESH)
                _sem_signal(acr_l, inc=1, device_id=(right,),
                            device_id_type=_MESH)
            gc_r = (my - t) % N_DEV
            gc_l = (my + t) % N_DEV
            dq_r[...] = ag_r[cur].astype(jnp.float32) * scale
            dq_l[...] = ag_l[cur].astype(jnp.float32) * scale
            st_r = pltpu.make_async_copy(dq_r, o_ref.at[r_rows(gc_r)],
                                         st_sem_r)
            st_l = pltpu.make_async_copy(dq_l, o_ref.at[l_rows(gc_l)],
                                         st_sem_l)
            st_r.start()
            st_l.start()
            st_r.wait()
            st_l.wait()

    f32, bf16, i8 = jnp.float32, jnp.bfloat16, jnp.int8
    return pl.pallas_call(
        body,
        out_shape=jax.ShapeDtypeStruct((M, N), f32),
        in_specs=[pl.BlockSpec(memory_space=pl.ANY)],
        out_specs=pl.BlockSpec(memory_space=pl.ANY),
        scratch_shapes=[
            pltpu.VMEM((2, HALF, N), f32),
            pltpu.VMEM((2, HALF, N), f32),
            pltpu.VMEM((2, HALF, N), f32),
            pltpu.VMEM((2, HALF, N), f32),
            pltpu.VMEM((2, HALF, N), bf16),
            pltpu.VMEM((2, HALF, N), bf16),
            pltpu.VMEM((2, HALF, N), bf16),
            pltpu.VMEM((2, HALF, N), bf16),
            pltpu.VMEM((2, HALF, N), i8),
            pltpu.VMEM((2, HALF, N), i8),
            pltpu.VMEM((HALF, N), i8),
            pltpu.VMEM((HALF, N), i8),
            pltpu.VMEM((HALF, N), f32),
            pltpu.VMEM((HALF, N), f32),
            pltpu.VMEM((6, 128), f32),
            pltpu.VMEM((6, 128), f32),
            pltpu.SemaphoreType.DMA((2,)),
            pltpu.SemaphoreType.DMA((2,)),
            pltpu.SemaphoreType.DMA((2,)),
            pltpu.SemaphoreType.DMA((2,)),
            pltpu.SemaphoreType.DMA((2,)),
            pltpu.SemaphoreType.DMA((2,)),
            pltpu.SemaphoreType.DMA((2,)),
            pltpu.SemaphoreType.DMA((2,)),
            pltpu.SemaphoreType.DMA,
            pltpu.SemaphoreType.DMA,
            pltpu.SemaphoreType.DMA,
            pltpu.SemaphoreType.DMA,
            pltpu.SemaphoreType.DMA((5,)),
            pltpu.SemaphoreType.DMA((5,)),
            pltpu.SemaphoreType.REGULAR,
            pltpu.SemaphoreType.REGULAR,
            pltpu.SemaphoreType.REGULAR,
            pltpu.SemaphoreType.REGULAR,
        ],
        compiler_params=pltpu.CompilerParams(
            collective_id=0, vmem_limit_bytes=50 * 1024 * 1024),
    )(partial)


def kernel(x, w_mat):
    partial = jnp.dot(x, w_mat, preferred_element_type=jnp.float32)
    return _fused_allreduce_quant(partial)


# device time: 877937 ns/iter; 3.8765x vs baseline; 1.5071x over previous
import jax
import jax.numpy as jnp
from jax import lax
from jax.experimental import pallas as pl
from jax.experimental.pallas import tpu as pltpu

N_DEV = 32
M = 4096
N = 8192
ROWS = M // N_DEV
HALF = ROWS // 2

_sem_signal = getattr(pl, "semaphore_signal", None) or pltpu.semaphore_signal
_sem_wait = getattr(pl, "semaphore_wait", None) or pltpu.semaphore_wait
_MESH = pl.DeviceIdType.MESH


def _ring_tables():
    mesh_order = []
    for z in range(4):
        for y in range(4):
            for x in ((0, 1) if y % 2 == 0 else (1, 0)):
                mesh_order.append((x, y, z))
    logical = {c: i for i, c in enumerate(mesh_order)}

    yz_path = []
    for z in range(4):
        for y in (range(4) if z % 2 == 0 else range(3, -1, -1)):
            yz_path.append((y, z))
    cycle = [(0, y, z) for (y, z) in yz_path]
    cycle += [(1, y, z) for (y, z) in reversed(yz_path)]

    pos = [0] * N_DEV
    right = [0] * N_DEV
    left = [0] * N_DEV
    for j, c in enumerate(cycle):
        lg = logical[c]
        pos[lg] = j
        right[lg] = logical[cycle[(j + 1) % N_DEV]]
        left[lg] = logical[cycle[(j - 1) % N_DEV]]
    return pos, right, left


def _fused_allreduce_quant(partial, pos, rgt, lft):
    def body(p_ref, pos_ref, rgt_ref, lft_ref, o_ref,
             local_r, local_l, acc_r, acc_l, sb_r, sb_l, rv_r, rv_l,
             ag_r, ag_l, oq_r, oq_l, dq_r, dq_l, aval, ainb,
             ss_r, rs_r, ss_l, rs_l,
             ag_ss_r, ag_rs_r, ag_ss_l, ag_rs_l,
             pf_sem_r, pf_sem_l, st_sem_r, st_sem_l,
             am_ss, am_rs,
             cr_r, cr_l, acr_r, acr_l):
        my = lax.axis_index("i")
        my_p = pos_ref[0]
        right = rgt_ref[0]
        left = lft_ref[0]

        def r_rows(c):
            return pl.ds(c * ROWS, HALF)

        def l_rows(c):
            return pl.ds(c * ROWS + HALF, HALF)

        barrier = pltpu.get_barrier_semaphore()
        for nbr in (left, right):
            _sem_signal(barrier, inc=1, device_id=(nbr,),
                        device_id_type=_MESH)
        _sem_wait(barrier, 2)

        init_r = pltpu.make_async_copy(p_ref.at[r_rows(my_p)],
                                       acc_r.at[0], pf_sem_r)
        init_l = pltpu.make_async_copy(p_ref.at[l_rows(my_p)],
                                       acc_l.at[0], pf_sem_l)
        init_r.start()
        init_l.start()
        init_r.wait()
        init_l.wait()
        sb_r[0] = acc_r[0].astype(jnp.bfloat16)
        sb_l[0] = acc_l[0].astype(jnp.bfloat16)

        for s in range(N_DEV - 1):
            cur, nxt = s % 2, (s + 1) % 2
            if s >= 2:
                _sem_wait(cr_r, 1)
                _sem_wait(cr_l, 1)
            rd_r = pltpu.make_async_remote_copy(
                src_ref=sb_r.at[cur], dst_ref=rv_r.at[cur],
                send_sem=ss_r.at[cur], recv_sem=rs_r.at[cur],
                device_id=(right,), device_id_type=_MESH)
            rd_l = pltpu.make_async_remote_copy(
                src_ref=sb_l.at[cur], dst_ref=rv_l.at[cur],
                send_sem=ss_l.at[cur], recv_sem=rs_l.at[cur],
                device_id=(left,), device_id_type=_MESH)
            rd_r.start()
            rd_l.start()

            rc_r = (my_p - s - 1) % N_DEV
            rc_l = (my_p + s + 1) % N_DEV
            pf_r = pltpu.make_async_copy(p_ref.at[r_rows(rc_r)],
                                         local_r.at[cur], pf_sem_r)
            pf_l = pltpu.make_async_copy(p_ref.at[l_rows(rc_l)],
                                         local_l.at[cur], pf_sem_l)
            pf_r.start()
            pf_l.start()

            rd_r.wait()
            rd_l.wait()
            pf_r.wait()
            pf_l.wait()

            acc_r[nxt] = rv_r[cur].astype(jnp.float32) + local_r[cur]
            acc_l[nxt] = rv_l[cur].astype(jnp.float32) + local_l[cur]
            if s <= N_DEV - 3:
                sb_r[nxt] = acc_r[nxt].astype(jnp.bfloat16)
                sb_l[nxt] = acc_l[nxt].astype(jnp.bfloat16)
            if s <= N_DEV - 4:
                _sem_signal(cr_r, inc=1, device_id=(left,),
                            device_id_type=_MESH)
                _sem_signal(cr_l, inc=1, device_id=(right,),
                            device_id_type=_MESH)


        am_local = jnp.maximum(jnp.max(jnp.abs(acc_r[1])),
                               jnp.max(jnp.abs(acc_l[1])))
        aval[pl.ds(0, 1)] = jnp.broadcast_to(am_local, (1, 128))
        for k in range(5):
            partner = my ^ (1 << k)
            rd = pltpu.make_async_remote_copy(
                src_ref=aval.at[pl.ds(k, 1)], dst_ref=ainb.at[pl.ds(k, 1)],
                send_sem=am_ss.at[k], recv_sem=am_rs.at[k],
                device_id=(partner,), device_id_type=_MESH)
            rd.start()
            rd.wait()
            aval[pl.ds(k + 1, 1)] = jnp.maximum(aval[pl.ds(k, 1)],
                                                ainb[pl.ds(k, 1)])
        scale = jnp.max(aval[pl.ds(5, 1)]) / 127.0

        oq_r[...] = jnp.clip(jnp.round(acc_r[1] / scale),
                             -127.0, 127.0).astype(jnp.int8)
        oq_l[...] = jnp.clip(jnp.round(acc_l[1] / scale),
                             -127.0, 127.0).astype(jnp.int8)
        dq_r[...] = oq_r[...].astype(jnp.float32) * scale
        dq_l[...] = oq_l[...].astype(jnp.float32) * scale
        st_r = pltpu.make_async_copy(dq_r, o_ref.at[r_rows((my_p + 1) % N_DEV)],
                                     st_sem_r)
        st_l = pltpu.make_async_copy(dq_l, o_ref.at[l_rows((my_p - 1) % N_DEV)],
                                     st_sem_l)
        st_r.start()
        st_l.start()
        st_r.wait()
        st_l.wait()

        for t in range(N_DEV - 1):
            cur = t % 2
            if t >= 2:
                _sem_wait(acr_r, 1)
                _sem_wait(acr_l, 1)
            src_r = oq_r if t == 0 else ag_r.at[(t - 1) % 2]
            src_l = oq_l if t == 0 else ag_l.at[(t - 1) % 2]
            rd_r = pltpu.make_async_remote_copy(
                src_ref=src_r, dst_ref=ag_r.at[cur],
                send_sem=ag_ss_r.at[cur], recv_sem=ag_rs_r.at[cur],
                device_id=(right,), device_id_type=_MESH)
            rd_l = pltpu.make_async_remote_copy(
                src_ref=src_l, dst_ref=ag_l.at[cur],
                send_sem=ag_ss_l.at[cur], recv_sem=ag_rs_l.at[cur],
                device_id=(left,), device_id_type=_MESH)
            rd_r.start()
            rd_l.start()
            rd_r.wait()
            rd_l.wait()
            if 1 <= t <= N_DEV - 3:
                _sem_signal(acr_r, inc=1, device_id=(left,),
                            device_id_type=_MESH)
                _sem_signal(acr_l, inc=1, device_id=(right,),
                            device_id_type=_MESH)
            gc_r = (my_p - t) % N_DEV
            gc_l = (my_p + t) % N_DEV
            dq_r[...] = ag_r[cur].astype(jnp.float32) * scale
            dq_l[...] = ag_l[cur].astype(jnp.float32) * scale
            st_r = pltpu.make_async_copy(dq_r, o_ref.at[r_rows(gc_r)],
                                         st_sem_r)
            st_l = pltpu.make_async_copy(dq_l, o_ref.at[l_rows(gc_l)],
                                         st_sem_l)
            st_r.start()
            st_l.start()
            st_r.wait()
            st_l.wait()

    f32, bf16, i8 = jnp.float32, jnp.bfloat16, jnp.int8
    return pl.pallas_call(
        body,
        out_shape=jax.ShapeDtypeStruct((M, N), f32),
        in_specs=[pl.BlockSpec(memory_space=pl.ANY),
                  pl.BlockSpec(memory_space=pltpu.SMEM),
                  pl.BlockSpec(memory_space=pltpu.SMEM),
                  pl.BlockSpec(memory_space=pltpu.SMEM)],
        out_specs=pl.BlockSpec(memory_space=pl.ANY),
        scratch_shapes=[
            pltpu.VMEM((2, HALF, N), f32),
            pltpu.VMEM((2, HALF, N), f32),
            pltpu.VMEM((2, HALF, N), f32),
            pltpu.VMEM((2, HALF, N), f32),
            pltpu.VMEM((2, HALF, N), bf16),
            pltpu.VMEM((2, HALF, N), bf16),
            pltpu.VMEM((2, HALF, N), bf16),
            pltpu.VMEM((2, HALF, N), bf16),
            pltpu.VMEM((2, HALF, N), i8),
            pltpu.VMEM((2, HALF, N), i8),
            pltpu.VMEM((HALF, N), i8),
            pltpu.VMEM((HALF, N), i8),
            pltpu.VMEM((HALF, N), f32),
            pltpu.VMEM((HALF, N), f32),
            pltpu.VMEM((6, 128), f32),
            pltpu.VMEM((6, 128), f32),
            pltpu.SemaphoreType.DMA((2,)),
            pltpu.SemaphoreType.DMA((2,)),
            pltpu.SemaphoreType.DMA((2,)),
            pltpu.SemaphoreType.DMA((2,)),
            pltpu.SemaphoreType.DMA((2,)),
            pltpu.SemaphoreType.DMA((2,)),
            pltpu.SemaphoreType.DMA((2,)),
            pltpu.SemaphoreType.DMA((2,)),
            pltpu.SemaphoreType.DMA,
            pltpu.SemaphoreType.DMA,
            pltpu.SemaphoreType.DMA,
            pltpu.SemaphoreType.DMA,
            pltpu.SemaphoreType.DMA((5,)),
            pltpu.SemaphoreType.DMA((5,)),
            pltpu.SemaphoreType.REGULAR,
            pltpu.SemaphoreType.REGULAR,
            pltpu.SemaphoreType.REGULAR,
            pltpu.SemaphoreType.REGULAR,
        ],
        compiler_params=pltpu.CompilerParams(
            collective_id=0, vmem_limit_bytes=50 * 1024 * 1024),
    )(partial, pos, rgt, lft)


def kernel(x, w_mat):
    partial = jnp.dot(x, w_mat, preferred_element_type=jnp.float32)
    pos_t, rgt_t, lft_t = _ring_tables()
    my = lax.axis_index("i")
    pos = jnp.asarray(pos_t, jnp.int32)[my][None]
    rgt = jnp.asarray(rgt_t, jnp.int32)[my][None]
    lft = jnp.asarray(lft_t, jnp.int32)[my][None]
    return _fused_allreduce_quant(partial, pos, rgt, lft)


# device time: 847935 ns/iter; 4.0137x vs baseline; 1.0354x over previous
import jax
import jax.numpy as jnp
from jax import lax
from jax.experimental import pallas as pl
from jax.experimental.pallas import tpu as pltpu

N_DEV = 32
M = 4096
N = 8192
ROWS = M // N_DEV
HALF = ROWS // 2

_sem_signal = getattr(pl, "semaphore_signal", None) or pltpu.semaphore_signal
_sem_wait = getattr(pl, "semaphore_wait", None) or pltpu.semaphore_wait
_MESH = pl.DeviceIdType.MESH


def _ring_tables():
    mesh_order = []
    for z in range(4):
        for y in range(4):
            for x in ((0, 1) if y % 2 == 0 else (1, 0)):
                mesh_order.append((x, y, z))
    logical = {c: i for i, c in enumerate(mesh_order)}

    yz_path = []
    for z in range(4):
        for y in (range(4) if z % 2 == 0 else range(3, -1, -1)):
            yz_path.append((y, z))
    cycle = [(0, y, z) for (y, z) in yz_path]
    cycle += [(1, y, z) for (y, z) in reversed(yz_path)]

    pos = [0] * N_DEV
    right = [0] * N_DEV
    left = [0] * N_DEV
    for j, c in enumerate(cycle):
        lg = logical[c]
        pos[lg] = j
        right[lg] = logical[cycle[(j + 1) % N_DEV]]
        left[lg] = logical[cycle[(j - 1) % N_DEV]]
    return pos, right, left


def _fused_allreduce_quant(partial, pos, rgt, lft):
    def body(p_ref, pos_ref, rgt_ref, lft_ref, o_ref,
             local_r, local_l, acc_r, acc_l, sb_r, sb_l, rv_r, rv_l,
             ag_r, ag_l, oq_r, oq_l, dq_r, dq_l, aval, ainb,
             ss_r, rs_r, ss_l, rs_l,
             ag_ss_r, ag_rs_r, ag_ss_l, ag_rs_l,
             pf_sem_r, pf_sem_l, st_sem_r, st_sem_l,
             am_ss, am_rs,
             cr_r, cr_l, acr_r, acr_l):
        my = lax.axis_index("i")
        my_p = pos_ref[0]
        right = rgt_ref[0]
        left = lft_ref[0]

        def r_rows(c):
            return pl.ds(c * ROWS, HALF)

        def l_rows(c):
            return pl.ds(c * ROWS + HALF, HALF)

        barrier = pltpu.get_barrier_semaphore()
        for nbr in (left, right):
            _sem_signal(barrier, inc=1, device_id=(nbr,),
                        device_id_type=_MESH)
        _sem_wait(barrier, 2)

        init_r = pltpu.make_async_copy(p_ref.at[r_rows(my_p)],
                                       acc_r.at[0], pf_sem_r)
        init_l = pltpu.make_async_copy(p_ref.at[l_rows(my_p)],
                                       acc_l.at[0], pf_sem_l)
        init_r.start()
        init_l.start()
        init_r.wait()
        init_l.wait()
        sb_r[0] = acc_r[0].astype(jnp.bfloat16)
        sb_l[0] = acc_l[0].astype(jnp.bfloat16)

        SUB = HALF // 2
        for s in range(N_DEV - 1):
            cur, nxt = s % 2, (s + 1) % 2
            if s >= 2:
                _sem_wait(cr_r, 1)
                _sem_wait(cr_l, 1)
            rds = []
            for (sb, rv, ss, rs, dev) in ((sb_r, rv_r, ss_r, rs_r, right),
                                          (sb_l, rv_l, ss_l, rs_l, left)):
                for sub in (0, 1):
                    rows = pl.ds(sub * SUB, SUB)
                    rds.append(pltpu.make_async_remote_copy(
                        src_ref=sb.at[cur, rows], dst_ref=rv.at[cur, rows],
                        send_sem=ss.at[2 * cur + sub],
                        recv_sem=rs.at[2 * cur + sub],
                        device_id=(dev,), device_id_type=_MESH))
            rd_ra, rd_rb, rd_la, rd_lb = rds
            rd_ra.start()
            rd_la.start()
            rd_rb.start()
            rd_lb.start()

            rc_r = (my_p - s - 1) % N_DEV
            rc_l = (my_p + s + 1) % N_DEV
            pf_r = pltpu.make_async_copy(p_ref.at[r_rows(rc_r)],
                                         local_r.at[cur], pf_sem_r)
            pf_l = pltpu.make_async_copy(p_ref.at[l_rows(rc_l)],
                                         local_l.at[cur], pf_sem_l)
            pf_r.start()
            pf_l.start()
            pf_r.wait()
            pf_l.wait()

            sA, sB = slice(0, SUB), slice(SUB, HALF)
            rd_ra.wait_recv()
            acc_r[nxt, sA] = (rv_r[cur, sA].astype(jnp.float32)
                              + local_r[cur, sA])
            if s <= N_DEV - 3:
                sb_r[nxt, sA] = acc_r[nxt, sA].astype(jnp.bfloat16)
            rd_la.wait_recv()
            acc_l[nxt, sA] = (rv_l[cur, sA].astype(jnp.float32)
                              + local_l[cur, sA])
            if s <= N_DEV - 3:
                sb_l[nxt, sA] = acc_l[nxt, sA].astype(jnp.bfloat16)
            rd_rb.wait_recv()
            acc_r[nxt, sB] = (rv_r[cur, sB].astype(jnp.float32)
                              + local_r[cur, sB])
            if s <= N_DEV - 3:
                sb_r[nxt, sB] = acc_r[nxt, sB].astype(jnp.bfloat16)
            rd_lb.wait_recv()
            acc_l[nxt, sB] = (rv_l[cur, sB].astype(jnp.float32)
                              + local_l[cur, sB])
            if s <= N_DEV - 3:
                sb_l[nxt, sB] = acc_l[nxt, sB].astype(jnp.bfloat16)

            rd_ra.wait_send()
            rd_la.wait_send()
            rd_rb.wait_send()
            rd_lb.wait_send()
            if s <= N_DEV - 4:
                _sem_signal(cr_r, inc=1, device_id=(left,),
                            device_id_type=_MESH)
                _sem_signal(cr_l, inc=1, device_id=(right,),
                            device_id_type=_MESH)


        am_local = jnp.maximum(jnp.max(jnp.abs(acc_r[1])),
                               jnp.max(jnp.abs(acc_l[1])))
        aval[pl.ds(0, 1)] = jnp.broadcast_to(am_local, (1, 128))
        for k in range(5):
            partner = my ^ (1 << k)
            rd = pltpu.make_async_remote_copy(
                src_ref=aval.at[pl.ds(k, 1)], dst_ref=ainb.at[pl.ds(k, 1)],
                send_sem=am_ss.at[k], recv_sem=am_rs.at[k],
                device_id=(partner,), device_id_type=_MESH)
            rd.start()
            rd.wait()
            aval[pl.ds(k + 1, 1)] = jnp.maximum(aval[pl.ds(k, 1)],
                                                ainb[pl.ds(k, 1)])
        scale = jnp.max(aval[pl.ds(5, 1)]) / 127.0

        oq_r[...] = jnp.clip(jnp.round(acc_r[1] / scale),
                             -127.0, 127.0).astype(jnp.int8)
        oq_l[...] = jnp.clip(jnp.round(acc_l[1] / scale),
                             -127.0, 127.0).astype(jnp.int8)
        dq_r[...] = oq_r[...].astype(jnp.float32) * scale
        dq_l[...] = oq_l[...].astype(jnp.float32) * scale
        st_r = pltpu.make_async_copy(dq_r, o_ref.at[r_rows((my_p + 1) % N_DEV)],
                                     st_sem_r)
        st_l = pltpu.make_async_copy(dq_l, o_ref.at[l_rows((my_p - 1) % N_DEV)],
                                     st_sem_l)
        st_r.start()
        st_l.start()
        st_r.wait()
        st_l.wait()

        for t in range(N_DEV - 1):
            cur = t % 2
            if t >= 2:
                _sem_wait(acr_r, 1)
                _sem_wait(acr_l, 1)
            rds = []
            for (oq, ag, ss, rs, dev) in (
                    (oq_r, ag_r, ag_ss_r, ag_rs_r, right),
                    (oq_l, ag_l, ag_ss_l, ag_rs_l, left)):
                for sub in (0, 1):
                    rows = pl.ds(sub * SUB, SUB)
                    src = (oq.at[rows] if t == 0
                           else ag.at[(t - 1) % 2, rows])
                    rds.append(pltpu.make_async_remote_copy(
                        src_ref=src, dst_ref=ag.at[cur, rows],
                        send_sem=ss.at[2 * cur + sub],
                        recv_sem=rs.at[2 * cur + sub],
                        device_id=(dev,), device_id_type=_MESH))
            rd_ra, rd_rb, rd_la, rd_lb = rds[0], rds[1], rds[2], rds[3]
            rd_ra.start()
            rd_la.start()
            rd_rb.start()
            rd_lb.start()

            gc_r = (my_p - t) % N_DEV
            gc_l = (my_p + t) % N_DEV
            sA, sB = slice(0, SUB), slice(SUB, HALF)
            rd_ra.wait_recv()
            dq_r[sA] = ag_r[cur, sA].astype(jnp.float32) * scale
            rd_la.wait_recv()
            dq_l[sA] = ag_l[cur, sA].astype(jnp.float32) * scale
            st_ra = pltpu.make_async_copy(
                dq_r.at[pl.ds(0, SUB)],
                o_ref.at[pl.ds(gc_r * ROWS, SUB)], st_sem_r)
            st_la = pltpu.make_async_copy(
                dq_l.at[pl.ds(0, SUB)],
                o_ref.at[pl.ds(gc_l * ROWS + HALF, SUB)], st_sem_l)
            st_ra.start()
            st_la.start()
            rd_rb.wait_recv()
            dq_r[sB] = ag_r[cur, sB].astype(jnp.float32) * scale
            rd_lb.wait_recv()
            dq_l[sB] = ag_l[cur, sB].astype(jnp.float32) * scale
            st_ra.wait()
            st_la.wait()
            st_rb = pltpu.make_async_copy(
                dq_r.at[pl.ds(SUB, SUB)],
                o_ref.at[pl.ds(gc_r * ROWS + SUB, SUB)], st_sem_r)
            st_lb = pltpu.make_async_copy(
                dq_l.at[pl.ds(SUB, SUB)],
                o_ref.at[pl.ds(gc_l * ROWS + HALF + SUB, SUB)], st_sem_l)
            st_rb.start()
            st_lb.start()
            rd_ra.wait_send()
            rd_la.wait_send()
            rd_rb.wait_send()
            rd_lb.wait_send()
            if 1 <= t <= N_DEV - 3:
                _sem_signal(acr_r, inc=1, device_id=(left,),
                            device_id_type=_MESH)
                _sem_signal(acr_l, inc=1, device_id=(right,),
                            device_id_type=_MESH)
            st_rb.wait()
            st_lb.wait()

    f32, bf16, i8 = jnp.float32, jnp.bfloat16, jnp.int8
    return pl.pallas_call(
        body,
        out_shape=jax.ShapeDtypeStruct((M, N), f32),
        in_specs=[pl.BlockSpec(memory_space=pl.ANY),
                  pl.BlockSpec(memory_space=pltpu.SMEM),
                  pl.BlockSpec(memory_space=pltpu.SMEM),
                  pl.BlockSpec(memory_space=pltpu.SMEM)],
        out_specs=pl.BlockSpec(memory_space=pl.ANY),
        scratch_shapes=[
            pltpu.VMEM((2, HALF, N), f32),
            pltpu.VMEM((2, HALF, N), f32),
            pltpu.VMEM((2, HALF, N), f32),
            pltpu.VMEM((2, HALF, N), f32),
            pltpu.VMEM((2, HALF, N), bf16),
            pltpu.VMEM((2, HALF, N), bf16),
            pltpu.VMEM((2, HALF, N), bf16),
            pltpu.VMEM((2, HALF, N), bf16),
            pltpu.VMEM((2, HALF, N), i8),
            pltpu.VMEM((2, HALF, N), i8),
            pltpu.VMEM((HALF, N), i8),
            pltpu.VMEM((HALF, N), i8),
            pltpu.VMEM((HALF, N), f32),
            pltpu.VMEM((HALF, N), f32),
            pltpu.VMEM((6, 128), f32),
            pltpu.VMEM((6, 128), f32),
            pltpu.SemaphoreType.DMA((4,)),
            pltpu.SemaphoreType.DMA((4,)),
            pltpu.SemaphoreType.DMA((4,)),
            pltpu.SemaphoreType.DMA((4,)),
            pltpu.SemaphoreType.DMA((4,)),
            pltpu.SemaphoreType.DMA((4,)),
            pltpu.SemaphoreType.DMA((4,)),
            pltpu.SemaphoreType.DMA((4,)),
            pltpu.SemaphoreType.DMA,
            pltpu.SemaphoreType.DMA,
            pltpu.SemaphoreType.DMA,
            pltpu.SemaphoreType.DMA,
            pltpu.SemaphoreType.DMA((5,)),
            pltpu.SemaphoreType.DMA((5,)),
            pltpu.SemaphoreType.REGULAR,
            pltpu.SemaphoreType.REGULAR,
            pltpu.SemaphoreType.REGULAR,
            pltpu.SemaphoreType.REGULAR,
        ],
        compiler_params=pltpu.CompilerParams(
            collective_id=0, vmem_limit_bytes=50 * 1024 * 1024),
    )(partial, pos, rgt, lft)


def kernel(x, w_mat):
    partial = jnp.dot(x, w_mat, preferred_element_type=jnp.float32)
    pos_t, rgt_t, lft_t = _ring_tables()
    my = lax.axis_index("i")
    pos = jnp.asarray(pos_t, jnp.int32)[my][None]
    rgt = jnp.asarray(rgt_t, jnp.int32)[my][None]
    lft = jnp.asarray(lft_t, jnp.int32)[my][None]
    return _fused_allreduce_quant(partial, pos, rgt, lft)


# device time: 847689 ns/iter; 4.0149x vs baseline; 1.0003x over previous
import jax
import jax.numpy as jnp
from jax import lax
from jax.experimental import pallas as pl
from jax.experimental.pallas import tpu as pltpu

N_DEV = 32
M = 4096
N = 8192
ROWS = M // N_DEV
HALF = ROWS // 2

_sem_signal = getattr(pl, "semaphore_signal", None) or pltpu.semaphore_signal
_sem_wait = getattr(pl, "semaphore_wait", None) or pltpu.semaphore_wait
_MESH = pl.DeviceIdType.MESH


def _ring_tables():
    mesh_order = []
    for z in range(4):
        for y in range(4):
            for x in ((0, 1) if y % 2 == 0 else (1, 0)):
                mesh_order.append((x, y, z))
    logical = {c: i for i, c in enumerate(mesh_order)}

    yz_path = []
    for z in range(4):
        for y in (range(4) if z % 2 == 0 else range(3, -1, -1)):
            yz_path.append((y, z))
    cycle = [(0, y, z) for (y, z) in yz_path]
    cycle += [(1, y, z) for (y, z) in reversed(yz_path)]

    pos = [0] * N_DEV
    right = [0] * N_DEV
    left = [0] * N_DEV
    for j, c in enumerate(cycle):
        lg = logical[c]
        pos[lg] = j
        right[lg] = logical[cycle[(j + 1) % N_DEV]]
        left[lg] = logical[cycle[(j - 1) % N_DEV]]
    return pos, right, left


def _fused_allreduce_quant(partial, pos, rgt, lft):
    def body(p_ref, pos_ref, rgt_ref, lft_ref, o_ref,
             local_r, local_l, acc_r, acc_l, sb_r, sb_l, rv_r, rv_l,
             ag_r, ag_l, oq_r, oq_l, dq_r, dq_l, aval, ainb,
             ss_r, rs_r, ss_l, rs_l,
             ag_ss_r, ag_rs_r, ag_ss_l, ag_rs_l,
             pf_sem_r, pf_sem_l, st_sem_r, st_sem_l,
             am_ss, am_rs,
             cr_r, cr_l, acr_r, acr_l):
        my = lax.axis_index("i")
        my_p = pos_ref[0]
        right = rgt_ref[0]
        left = lft_ref[0]

        def r_rows(c):
            return pl.ds(c * ROWS, HALF)

        def l_rows(c):
            return pl.ds(c * ROWS + HALF, HALF)

        barrier = pltpu.get_barrier_semaphore()
        for nbr in (left, right):
            _sem_signal(barrier, inc=1, device_id=(nbr,),
                        device_id_type=_MESH)
        _sem_wait(barrier, 2)

        init_r = pltpu.make_async_copy(p_ref.at[r_rows(my_p)],
                                       acc_r.at[0], pf_sem_r)
        init_l = pltpu.make_async_copy(p_ref.at[l_rows(my_p)],
                                       acc_l.at[0], pf_sem_l)
        init_r.start()
        init_l.start()
        init_r.wait()
        init_l.wait()
        sb_r[0] = acc_r[0].astype(jnp.bfloat16)
        sb_l[0] = acc_l[0].astype(jnp.bfloat16)

        SUB = HALF // 2
        rds_m1 = rds_m2 = None
        for s in range(N_DEV - 1):
            cur3, nx3 = s % 3, (s + 1) % 3
            cur, nxt = s % 2, (s + 1) % 2
            if s >= 3:
                _sem_wait(cr_r, 1)
                _sem_wait(cr_l, 1)
            rds = []
            for (sb, rv, ss, rs, dev) in ((sb_r, rv_r, ss_r, rs_r, right),
                                          (sb_l, rv_l, ss_l, rs_l, left)):
                for sub in (0, 1):
                    rows = pl.ds(sub * SUB, SUB)
                    rds.append(pltpu.make_async_remote_copy(
                        src_ref=sb.at[cur3, rows], dst_ref=rv.at[cur3, rows],
                        send_sem=ss.at[2 * cur3 + sub],
                        recv_sem=rs.at[2 * cur3 + sub],
                        device_id=(dev,), device_id_type=_MESH))
            rd_ra, rd_rb, rd_la, rd_lb = rds
            rd_ra.start()
            rd_la.start()
            rd_rb.start()
            rd_lb.start()
            if rds_m2 is not None:
                for d in rds_m2:
                    d.wait_send()

            rc_r = (my_p - s - 1) % N_DEV
            rc_l = (my_p + s + 1) % N_DEV
            pf_r = pltpu.make_async_copy(p_ref.at[r_rows(rc_r)],
                                         local_r.at[cur], pf_sem_r)
            pf_l = pltpu.make_async_copy(p_ref.at[l_rows(rc_l)],
                                         local_l.at[cur], pf_sem_l)
            pf_r.start()
            pf_l.start()
            pf_r.wait()
            pf_l.wait()

            sA, sB = slice(0, SUB), slice(SUB, HALF)
            rd_ra.wait_recv()
            acc_r[nxt, sA] = (rv_r[cur3, sA].astype(jnp.float32)
                              + local_r[cur, sA])
            if s <= N_DEV - 3:
                sb_r[nx3, sA] = acc_r[nxt, sA].astype(jnp.bfloat16)
            rd_la.wait_recv()
            acc_l[nxt, sA] = (rv_l[cur3, sA].astype(jnp.float32)
                              + local_l[cur, sA])
            if s <= N_DEV - 3:
                sb_l[nx3, sA] = acc_l[nxt, sA].astype(jnp.bfloat16)
            rd_rb.wait_recv()
            acc_r[nxt, sB] = (rv_r[cur3, sB].astype(jnp.float32)
                              + local_r[cur, sB])
            if s <= N_DEV - 3:
                sb_r[nx3, sB] = acc_r[nxt, sB].astype(jnp.bfloat16)
            rd_lb.wait_recv()
            acc_l[nxt, sB] = (rv_l[cur3, sB].astype(jnp.float32)
                              + local_l[cur, sB])
            if s <= N_DEV - 3:
                sb_l[nx3, sB] = acc_l[nxt, sB].astype(jnp.bfloat16)

            if s <= N_DEV - 5:
                _sem_signal(cr_r, inc=1, device_id=(left,),
                            device_id_type=_MESH)
                _sem_signal(cr_l, inc=1, device_id=(right,),
                            device_id_type=_MESH)
            rds_m2, rds_m1 = rds_m1, rds
        for grp in (rds_m2, rds_m1):
            for d in grp:
                d.wait_send()


        am_local = jnp.maximum(jnp.max(jnp.abs(acc_r[1])),
                               jnp.max(jnp.abs(acc_l[1])))
        aval[pl.ds(0, 1)] = jnp.broadcast_to(am_local, (1, 128))
        for k in range(5):
            partner = my ^ (1 << k)
            rd = pltpu.make_async_remote_copy(
                src_ref=aval.at[pl.ds(k, 1)], dst_ref=ainb.at[pl.ds(k, 1)],
                send_sem=am_ss.at[k], recv_sem=am_rs.at[k],
                device_id=(partner,), device_id_type=_MESH)
            rd.start()
            rd.wait()
            aval[pl.ds(k + 1, 1)] = jnp.maximum(aval[pl.ds(k, 1)],
                                                ainb[pl.ds(k, 1)])
        scale = jnp.max(aval[pl.ds(5, 1)]) / 127.0

        oq_r[...] = jnp.clip(jnp.round(acc_r[1] / scale),
                             -127.0, 127.0).astype(jnp.int8)
        oq_l[...] = jnp.clip(jnp.round(acc_l[1] / scale),
                             -127.0, 127.0).astype(jnp.int8)
        dq_r[...] = oq_r[...].astype(jnp.float32) * scale
        dq_l[...] = oq_l[...].astype(jnp.float32) * scale
        st_r = pltpu.make_async_copy(dq_r, o_ref.at[r_rows((my_p + 1) % N_DEV)],
                                     st_sem_r)
        st_l = pltpu.make_async_copy(dq_l, o_ref.at[l_rows((my_p - 1) % N_DEV)],
                                     st_sem_l)
        st_r.start()
        st_l.start()
        st_r.wait()
        st_l.wait()

        for t in range(N_DEV - 1):
            cur = t % 3
            if t >= 3:
                _sem_wait(acr_r, 1)
                _sem_wait(acr_l, 1)
            rds = []
            for (oq, ag, ss, rs, dev) in (
                    (oq_r, ag_r, ag_ss_r, ag_rs_r, right),
                    (oq_l, ag_l, ag_ss_l, ag_rs_l, left)):
                for sub in (0, 1):
                    rows = pl.ds(sub * SUB, SUB)
                    src = (oq.at[rows] if t == 0
                           else ag.at[(t - 1) % 3, rows])
                    rds.append(pltpu.make_async_remote_copy(
                        src_ref=src, dst_ref=ag.at[cur, rows],
                        send_sem=ss.at[2 * cur + sub],
                        recv_sem=rs.at[2 * cur + sub],
                        device_id=(dev,), device_id_type=_MESH))
            rd_ra, rd_rb, rd_la, rd_lb = rds[0], rds[1], rds[2], rds[3]
            rd_ra.start()
            rd_la.start()
            rd_rb.start()
            rd_lb.start()

            gc_r = (my_p - t) % N_DEV
            gc_l = (my_p + t) % N_DEV
            sA, sB = slice(0, SUB), slice(SUB, HALF)
            rd_ra.wait_recv()
            dq_r[sA] = ag_r[cur, sA].astype(jnp.float32) * scale
            rd_la.wait_recv()
            dq_l[sA] = ag_l[cur, sA].astype(jnp.float32) * scale
            st_ra = pltpu.make_async_copy(
                dq_r.at[pl.ds(0, SUB)],
                o_ref.at[pl.ds(gc_r * ROWS, SUB)], st_sem_r)
            st_la = pltpu.make_async_copy(
                dq_l.at[pl.ds(0, SUB)],
                o_ref.at[pl.ds(gc_l * ROWS + HALF, SUB)], st_sem_l)
            st_ra.start()
            st_la.start()
            rd_rb.wait_recv()
            dq_r[sB] = ag_r[cur, sB].astype(jnp.float32) * scale
            rd_lb.wait_recv()
            dq_l[sB] = ag_l[cur, sB].astype(jnp.float32) * scale
            st_ra.wait()
            st_la.wait()
            st_rb = pltpu.make_async_copy(
                dq_r.at[pl.ds(SUB, SUB)],
                o_ref.at[pl.ds(gc_r * ROWS + SUB, SUB)], st_sem_r)
            st_lb = pltpu.make_async_copy(
                dq_l.at[pl.ds(SUB, SUB)],
                o_ref.at[pl.ds(gc_l * ROWS + HALF + SUB, SUB)], st_sem_l)
            st_rb.start()
            st_lb.start()
            rd_ra.wait_send()
            rd_la.wait_send()
            rd_rb.wait_send()
            rd_lb.wait_send()
            if 1 <= t <= N_DEV - 4:
                _sem_signal(acr_r, inc=1, device_id=(left,),
                            device_id_type=_MESH)
                _sem_signal(acr_l, inc=1, device_id=(right,),
                            device_id_type=_MESH)
            st_rb.wait()
            st_lb.wait()

    f32, bf16, i8 = jnp.float32, jnp.bfloat16, jnp.int8
    return pl.pallas_call(
        body,
        out_shape=jax.ShapeDtypeStruct((M, N), f32),
        in_specs=[pl.BlockSpec(memory_space=pl.ANY),
                  pl.BlockSpec(memory_space=pltpu.SMEM),
                  pl.BlockSpec(memory_space=pltpu.SMEM),
                  pl.BlockSpec(memory_space=pltpu.SMEM)],
        out_specs=pl.BlockSpec(memory_space=pl.ANY),
        scratch_shapes=[
            pltpu.VMEM((2, HALF, N), f32),
            pltpu.VMEM((2, HALF, N), f32),
            pltpu.VMEM((2, HALF, N), f32),
            pltpu.VMEM((2, HALF, N), f32),
            pltpu.VMEM((3, HALF, N), bf16),
            pltpu.VMEM((3, HALF, N), bf16),
            pltpu.VMEM((3, HALF, N), bf16),
            pltpu.VMEM((3, HALF, N), bf16),
            pltpu.VMEM((3, HALF, N), i8),
            pltpu.VMEM((3, HALF, N), i8),
            pltpu.VMEM((HALF, N), i8),
            pltpu.VMEM((HALF, N), i8),
            pltpu.VMEM((HALF, N), f32),
            pltpu.VMEM((HALF, N), f32),
            pltpu.VMEM((6, 128), f32),
            pltpu.VMEM((6, 128), f32),
            pltpu.SemaphoreType.DMA((6,)),
            pltpu.SemaphoreType.DMA((6,)),
            pltpu.SemaphoreType.DMA((6,)),
            pltpu.SemaphoreType.DMA((6,)),
            pltpu.SemaphoreType.DMA((6,)),
            pltpu.SemaphoreType.DMA((6,)),
            pltpu.SemaphoreType.DMA((6,)),
            pltpu.SemaphoreType.DMA((6,)),
            pltpu.SemaphoreType.DMA,
            pltpu.SemaphoreType.DMA,
            pltpu.SemaphoreType.DMA,
            pltpu.SemaphoreType.DMA,
            pltpu.SemaphoreType.DMA((5,)),
            pltpu.SemaphoreType.DMA((5,)),
            pltpu.SemaphoreType.REGULAR,
            pltpu.SemaphoreType.REGULAR,
            pltpu.SemaphoreType.REGULAR,
            pltpu.SemaphoreType.REGULAR,
        ],
        compiler_params=pltpu.CompilerParams(
            collective_id=0, vmem_limit_bytes=50 * 1024 * 1024),
    )(partial, pos, rgt, lft)


def kernel(x, w_mat):
    partial = jnp.dot(x, w_mat, preferred_element_type=jnp.float32)
    pos_t, rgt_t, lft_t = _ring_tables()
    my = lax.axis_index("i")
    pos = jnp.asarray(pos_t, jnp.int32)[my][None]
    rgt = jnp.asarray(rgt_t, jnp.int32)[my][None]
    lft = jnp.asarray(lft_t, jnp.int32)[my][None]
    return _fused_allreduce_quant(partial, pos, rgt, lft)


# device time: 840296 ns/iter; 4.0502x vs baseline; 1.0088x over previous
import jax
import jax.numpy as jnp
from jax import lax
from jax.experimental import pallas as pl
from jax.experimental.pallas import tpu as pltpu

N_DEV = 32
M = 4096
N = 8192
ROWS = M // N_DEV
HALF = ROWS // 2

_sem_signal = getattr(pl, "semaphore_signal", None) or pltpu.semaphore_signal
_sem_wait = getattr(pl, "semaphore_wait", None) or pltpu.semaphore_wait
_MESH = pl.DeviceIdType.MESH


def _ring_tables():
    mesh_order = []
    for z in range(4):
        for y in range(4):
            for x in ((0, 1) if y % 2 == 0 else (1, 0)):
                mesh_order.append((x, y, z))
    logical = {c: i for i, c in enumerate(mesh_order)}

    yz_path = []
    for z in range(4):
        for y in (range(4) if z % 2 == 0 else range(3, -1, -1)):
            yz_path.append((y, z))
    cycle = [(0, y, z) for (y, z) in yz_path]
    cycle += [(1, y, z) for (y, z) in reversed(yz_path)]

    pos = [0] * N_DEV
    right = [0] * N_DEV
    left = [0] * N_DEV
    for j, c in enumerate(cycle):
        lg = logical[c]
        pos[lg] = j
        right[lg] = logical[cycle[(j + 1) % N_DEV]]
        left[lg] = logical[cycle[(j - 1) % N_DEV]]
    return pos, right, left


def _fused_allreduce_quant(partial, pos, rgt, lft):
    def body(p_ref, pos_ref, rgt_ref, lft_ref, o_ref,
             local_r, local_l, acc_r, acc_l, sb_r, sb_l, rv_r, rv_l,
             ag_r, ag_l, oq_r, oq_l, dq_r, dq_l, aval, ainb,
             ss_r, rs_r, ss_l, rs_l,
             ag_ss_r, ag_rs_r, ag_ss_l, ag_rs_l,
             pf_sem_r, pf_sem_l, st_sem_r, st_sem_l,
             am_ss, am_rs,
             cr_r, cr_l, acr_r, acr_l):
        my = lax.axis_index("i")
        my_p = pos_ref[0]
        right = rgt_ref[0]
        left = lft_ref[0]

        def r_rows(c):
            return pl.ds(c * ROWS, HALF)

        def l_rows(c):
            return pl.ds(c * ROWS + HALF, HALF)

        barrier = pltpu.get_barrier_semaphore()
        for nbr in (left, right):
            _sem_signal(barrier, inc=1, device_id=(nbr,),
                        device_id_type=_MESH)
        _sem_wait(barrier, 2)

        init_r = pltpu.make_async_copy(p_ref.at[r_rows(my_p)],
                                       acc_r.at[0], pf_sem_r)
        init_l = pltpu.make_async_copy(p_ref.at[l_rows(my_p)],
                                       acc_l.at[0], pf_sem_l)
        init_r.start()
        init_l.start()
        init_r.wait()
        init_l.wait()
        sb_r[0] = acc_r[0].astype(jnp.bfloat16)
        sb_l[0] = acc_l[0].astype(jnp.bfloat16)

        SUB = HALF // 2
        rds_m1 = rds_m2 = None
        for s in range(N_DEV - 1):
            cur3, nx3 = s % 3, (s + 1) % 3
            cur, nxt = s % 2, (s + 1) % 2
            if s >= 3:
                _sem_wait(cr_r, 1)
                _sem_wait(cr_l, 1)
            rds = []
            for (sb, rv, ss, rs, dev) in ((sb_r, rv_r, ss_r, rs_r, right),
                                          (sb_l, rv_l, ss_l, rs_l, left)):
                for sub in (0, 1):
                    rows = pl.ds(sub * SUB, SUB)
                    rds.append(pltpu.make_async_remote_copy(
                        src_ref=sb.at[cur3, rows], dst_ref=rv.at[cur3, rows],
                        send_sem=ss.at[2 * cur3 + sub],
                        recv_sem=rs.at[2 * cur3 + sub],
                        device_id=(dev,), device_id_type=_MESH))
            rd_ra, rd_rb, rd_la, rd_lb = rds
            rd_ra.start()
            rd_la.start()
            rd_rb.start()
            rd_lb.start()
            if rds_m2 is not None:
                for d in rds_m2:
                    d.wait_send()

            rc_r = (my_p - s - 1) % N_DEV
            rc_l = (my_p + s + 1) % N_DEV
            pf_r = pltpu.make_async_copy(p_ref.at[r_rows(rc_r)],
                                         local_r.at[cur], pf_sem_r)
            pf_l = pltpu.make_async_copy(p_ref.at[l_rows(rc_l)],
                                         local_l.at[cur], pf_sem_l)
            pf_r.start()
            pf_l.start()
            pf_r.wait()
            pf_l.wait()

            sA, sB = slice(0, SUB), slice(SUB, HALF)
            last = s == N_DEV - 2
            rd_ra.wait_recv()
            sum_ra = rv_r[cur3, sA].astype(jnp.float32) + local_r[cur, sA]
            if last:
                acc_r[nxt, sA] = sum_ra
            else:
                sb_r[nx3, sA] = sum_ra.astype(jnp.bfloat16)
            rd_la.wait_recv()
            sum_la = rv_l[cur3, sA].astype(jnp.float32) + local_l[cur, sA]
            if last:
                acc_l[nxt, sA] = sum_la
            else:
                sb_l[nx3, sA] = sum_la.astype(jnp.bfloat16)
            rd_rb.wait_recv()
            sum_rb = rv_r[cur3, sB].astype(jnp.float32) + local_r[cur, sB]
            if last:
                acc_r[nxt, sB] = sum_rb
            else:
                sb_r[nx3, sB] = sum_rb.astype(jnp.bfloat16)
            rd_lb.wait_recv()
            sum_lb = rv_l[cur3, sB].astype(jnp.float32) + local_l[cur, sB]
            if last:
                acc_l[nxt, sB] = sum_lb
            else:
                sb_l[nx3, sB] = sum_lb.astype(jnp.bfloat16)

            if s <= N_DEV - 5:
                _sem_signal(cr_r, inc=1, device_id=(left,),
                            device_id_type=_MESH)
                _sem_signal(cr_l, inc=1, device_id=(right,),
                            device_id_type=_MESH)
            rds_m2, rds_m1 = rds_m1, rds
        for grp in (rds_m2, rds_m1):
            for d in grp:
                d.wait_send()


        am_local = jnp.maximum(jnp.max(jnp.abs(acc_r[1])),
                               jnp.max(jnp.abs(acc_l[1])))
        aval[pl.ds(0, 1)] = jnp.broadcast_to(am_local, (1, 128))
        for k in range(5):
            partner = my ^ (1 << k)
            rd = pltpu.make_async_remote_copy(
                src_ref=aval.at[pl.ds(k, 1)], dst_ref=ainb.at[pl.ds(k, 1)],
                send_sem=am_ss.at[k], recv_sem=am_rs.at[k],
                device_id=(partner,), device_id_type=_MESH)
            rd.start()
            rd.wait()
            aval[pl.ds(k + 1, 1)] = jnp.maximum(aval[pl.ds(k, 1)],
                                                ainb[pl.ds(k, 1)])
        scale = jnp.max(aval[pl.ds(5, 1)]) / 127.0

        oq_r[...] = jnp.clip(jnp.round(acc_r[1] / scale),
                             -127.0, 127.0).astype(jnp.int8)
        oq_l[...] = jnp.clip(jnp.round(acc_l[1] / scale),
                             -127.0, 127.0).astype(jnp.int8)
        dq_r[...] = oq_r[...].astype(jnp.float32) * scale
        dq_l[...] = oq_l[...].astype(jnp.float32) * scale
        st_r = pltpu.make_async_copy(dq_r, o_ref.at[r_rows((my_p + 1) % N_DEV)],
                                     st_sem_r)
        st_l = pltpu.make_async_copy(dq_l, o_ref.at[l_rows((my_p - 1) % N_DEV)],
                                     st_sem_l)
        st_r.start()
        st_l.start()
        st_r.wait()
        st_l.wait()

        for t in range(N_DEV - 1):
            cur = t % 3
            if t >= 3:
                _sem_wait(acr_r, 1)
                _sem_wait(acr_l, 1)
            rds = []
            for (oq, ag, ss, rs, dev) in (
                    (oq_r, ag_r, ag_ss_r, ag_rs_r, right),
                    (oq_l, ag_l, ag_ss_l, ag_rs_l, left)):
                for sub in (0, 1):
                    rows = pl.ds(sub * SUB, SUB)
                    src = (oq.at[rows] if t == 0
                           else ag.at[(t - 1) % 3, rows])
                    rds.append(pltpu.make_async_remote_copy(
                        src_ref=src, dst_ref=ag.at[cur, rows],
                        send_sem=ss.at[2 * cur + sub],
                        recv_sem=rs.at[2 * cur + sub],
                        device_id=(dev,), device_id_type=_MESH))
            rd_ra, rd_rb, rd_la, rd_lb = rds[0], rds[1], rds[2], rds[3]
            rd_ra.start()
            rd_la.start()
            rd_rb.start()
            rd_lb.start()

            gc_r = (my_p - t) % N_DEV
            gc_l = (my_p + t) % N_DEV
            sA, sB = slice(0, SUB), slice(SUB, HALF)
            rd_ra.wait_recv()
            dq_r[sA] = ag_r[cur, sA].astype(jnp.float32) * scale
            rd_la.wait_recv()
            dq_l[sA] = ag_l[cur, sA].astype(jnp.float32) * scale
            st_ra = pltpu.make_async_copy(
                dq_r.at[pl.ds(0, SUB)],
                o_ref.at[pl.ds(gc_r * ROWS, SUB)], st_sem_r)
            st_la = pltpu.make_async_copy(
                dq_l.at[pl.ds(0, SUB)],
                o_ref.at[pl.ds(gc_l * ROWS + HALF, SUB)], st_sem_l)
            st_ra.start()
            st_la.start()
            rd_rb.wait_recv()
            dq_r[sB] = ag_r[cur, sB].astype(jnp.float32) * scale
            rd_lb.wait_recv()
            dq_l[sB] = ag_l[cur, sB].astype(jnp.float32) * scale
            st_ra.wait()
            st_la.wait()
            st_rb = pltpu.make_async_copy(
                dq_r.at[pl.ds(SUB, SUB)],
                o_ref.at[pl.ds(gc_r * ROWS + SUB, SUB)], st_sem_r)
            st_lb = pltpu.make_async_copy(
                dq_l.at[pl.ds(SUB, SUB)],
                o_ref.at[pl.ds(gc_l * ROWS + HALF + SUB, SUB)], st_sem_l)
            st_rb.start()
            st_lb.start()
            rd_ra.wait_send()
            rd_la.wait_send()
            rd_rb.wait_send()
            rd_lb.wait_send()
            if 1 <= t <= N_DEV - 4:
                _sem_signal(acr_r, inc=1, device_id=(left,),
                            device_id_type=_MESH)
                _sem_signal(acr_l, inc=1, device_id=(right,),
                            device_id_type=_MESH)
            st_rb.wait()
            st_lb.wait()

    f32, bf16, i8 = jnp.float32, jnp.bfloat16, jnp.int8
    return pl.pallas_call(
        body,
        out_shape=jax.ShapeDtypeStruct((M, N), f32),
        in_specs=[pl.BlockSpec(memory_space=pl.ANY),
                  pl.BlockSpec(memory_space=pltpu.SMEM),
                  pl.BlockSpec(memory_space=pltpu.SMEM),
                  pl.BlockSpec(memory_space=pltpu.SMEM)],
        out_specs=pl.BlockSpec(memory_space=pl.ANY),
        scratch_shapes=[
            pltpu.VMEM((2, HALF, N), f32),
            pltpu.VMEM((2, HALF, N), f32),
            pltpu.VMEM((2, HALF, N), f32),
            pltpu.VMEM((2, HALF, N), f32),
            pltpu.VMEM((3, HALF, N), bf16),
            pltpu.VMEM((3, HALF, N), bf16),
            pltpu.VMEM((3, HALF, N), bf16),
            pltpu.VMEM((3, HALF, N), bf16),
            pltpu.VMEM((3, HALF, N), i8),
            pltpu.VMEM((3, HALF, N), i8),
            pltpu.VMEM((HALF, N), i8),
            pltpu.VMEM((HALF, N), i8),
            pltpu.VMEM((HALF, N), f32),
            pltpu.VMEM((HALF, N), f32),
            pltpu.VMEM((6, 128), f32),
            pltpu.VMEM((6, 128), f32),
            pltpu.SemaphoreType.DMA((6,)),
            pltpu.SemaphoreType.DMA((6,)),
            pltpu.SemaphoreType.DMA((6,)),
            pltpu.SemaphoreType.DMA((6,)),
            pltpu.SemaphoreType.DMA((6,)),
            pltpu.SemaphoreType.DMA((6,)),
            pltpu.SemaphoreType.DMA((6,)),
            pltpu.SemaphoreType.DMA((6,)),
            pltpu.SemaphoreType.DMA,
            pltpu.SemaphoreType.DMA,
            pltpu.SemaphoreType.DMA,
            pltpu.SemaphoreType.DMA,
            pltpu.SemaphoreType.DMA((5,)),
            pltpu.SemaphoreType.DMA((5,)),
            pltpu.SemaphoreType.REGULAR,
            pltpu.SemaphoreType.REGULAR,
            pltpu.SemaphoreType.REGULAR,
            pltpu.SemaphoreType.REGULAR,
        ],
        compiler_params=pltpu.CompilerParams(
            collective_id=0, vmem_limit_bytes=50 * 1024 * 1024),
    )(partial, pos, rgt, lft)


def kernel(x, w_mat):
    partial = jnp.dot(x, w_mat, preferred_element_type=jnp.float32)
    pos_t, rgt_t, lft_t = _ring_tables()
    my = lax.axis_index("i")
    pos = jnp.asarray(pos_t, jnp.int32)[my][None]
    rgt = jnp.asarray(rgt_t, jnp.int32)[my][None]
    lft = jnp.asarray(lft_t, jnp.int32)[my][None]
    return _fused_allreduce_quant(partial, pos, rgt, lft)
